# baseline (speedup 1.0000x reference)
import functools

import jax
import jax.numpy as jnp
from jax import lax
from jax.experimental import pallas as pl
from jax.experimental.pallas import tpu as pltpu
from jax.experimental.pallas import tpu_sc as plsc

F32 = jnp.float32

NC = 2
NS = 16

CG = 80


def _silu(v):
    return v * jax.nn.sigmoid(v)


def _tables_body(x_ref, wd_ref, ws_ref, td_ref, ts_ref):
    xb = x_ref[...]
    td_ref[...] = jnp.dot(xb, wd_ref[...], preferred_element_type=F32)
    ts_ref[...] = jnp.dot(xb, ws_ref[...], preferred_element_type=F32)


def _make_tables_call(np_, d, h, bn):
    nb = np_ // bn
    return pl.pallas_call(
        _tables_body,
        grid=(nb,),
        in_specs=[
            pl.BlockSpec((bn, d), lambda i: (i, 0)),
            pl.BlockSpec((d, h), lambda i: (0, 0)),
            pl.BlockSpec((d, h), lambda i: (0, 0)),
        ],
        out_specs=[
            pl.BlockSpec((bn, h), lambda i: (i, 0)),
            pl.BlockSpec((bn, h), lambda i: (i, 0)),
        ],
        out_shape=[
            jax.ShapeDtypeStruct((np_, h), F32),
            jax.ShapeDtypeStruct((np_, h), F32),
        ],
    )


def _make_gather_call(np_, e, h):
    epw = e // (NC * NS)
    nchunks = epw // CG
    mesh = plsc.VectorSubcoreMesh(core_axis_name="c", subcore_axis_name="s")

    @functools.partial(
        pl.kernel,
        mesh=mesh,
        compiler_params=pltpu.CompilerParams(needs_layout_passes=False),
        out_type=(
            jax.ShapeDtypeStruct((e, h), F32),
            jax.ShapeDtypeStruct((e, h), F32),
            jax.ShapeDtypeStruct((e, 16), F32),
        ),
        scratch_types=[
            pltpu.VMEM((CG,), jnp.int32),
            pltpu.VMEM((CG,), jnp.int32),
            pltpu.VMEM((CG, h), F32),
            pltpu.VMEM((CG, h), F32),
            pltpu.VMEM((CG, 16), F32),
            pltpu.VMEM((np_ * 4,), F32),
            pltpu.SemaphoreType.DMA,
            pltpu.SemaphoreType.DMA,
        ],
    )
    def gather_k(td, ts, p4, dstv, srcv, gd, gs, grel,
                 idxd, idxs, bxd, bxs, brel, post, s1, s2):
        cid = lax.axis_index("c")
        sid = lax.axis_index("s")
        wid = sid * NC + cid
        base = pl.multiple_of(wid * epw, 8)
        pltpu.sync_copy(p4, post)

        def zrow(i, carry):
            brel[i] = jnp.zeros((16,), F32)
            return carry

        lax.fori_loop(0, CG, zrow, 0)

        def chunk(i, carry):
            off = pl.multiple_of(base + i * CG, 8)
            pltpu.sync_copy(dstv.at[pl.ds(off, CG)], idxd)
            pltpu.sync_copy(srcv.at[pl.ds(off, CG)], idxs)
            a = pltpu.async_copy(td.at[idxd], bxd, s1)
            b = pltpu.async_copy(ts.at[idxs], bxs, s2)

            def group(g, carry2):
                idv = idxd[pl.ds(g * 16, 16)]
                isv = idxs[pl.ds(g * 16, 16)]
                lane = lax.iota(jnp.int32, 16)
                row = g * 16 + lane
                for comp in range(3):
                    cvec = jnp.full((16,), comp, jnp.int32)
                    pd = plsc.load_gather(post, [idv * 4 + comp])
                    ps = plsc.load_gather(post, [isv * 4 + comp])
                    plsc.store_scatter(brel, [row, cvec], pd - ps)
                return carry2

            lax.fori_loop(0, CG // 16, group, 0)
            a.wait()
            b.wait()
            pltpu.sync_copy(bxd, gd.at[pl.ds(off, CG)])
            pltpu.sync_copy(bxs, gs.at[pl.ds(off, CG)])
            pltpu.sync_copy(brel, grel.at[pl.ds(off, CG)])
            return carry

        lax.fori_loop(0, nchunks, chunk, 0)

    return gather_k


def _edge_body(emit_m, gd_ref, gs_ref, rel_ref, ea_ref,
               wd2_ref, we_ref, be1_ref, we2_ref, be2_ref,
               wc1_ref, bc1_ref, wc2_ref, bc2_ref, *out_refs):
    diff = rel_ref[...]
    d2 = jnp.sum(diff * diff, axis=1, keepdims=True)
    pre = (gd_ref[...] + gs_ref[...] + d2 * wd2_ref[...]
           + jnp.dot(ea_ref[...], we_ref[...], preferred_element_type=F32)
           + be1_ref[...])
    m = _silu(pre)
    m2 = _silu(jnp.dot(m, we2_ref[...], preferred_element_type=F32)
               + be2_ref[...])
    cc = _silu(jnp.dot(m2, wc1_ref[...], preferred_element_type=F32)
               + bc1_ref[...])
    c = jnp.sum(cc * wc2_ref[...], axis=1, keepdims=True) + bc2_ref[...]
    cb = jnp.broadcast_to(c, diff.shape)
    if emit_m:
        out_refs[0][...] = m2
        out_refs[1][...] = cb
    else:
        out_refs[0][...] = cb


def _make_edge_call(e, h, ed, be, emit_m):
    nb = e // be
    wspec = lambda r, c: pl.BlockSpec((r, c), lambda i: (0, 0))
    out_specs = [pl.BlockSpec((be, 16), lambda i: (i, 0))]
    out_shape = [jax.ShapeDtypeStruct((e, 16), F32)]
    if emit_m:
        out_specs = [pl.BlockSpec((be, h), lambda i: (i, 0))] + out_specs
        out_shape = [jax.ShapeDtypeStruct((e, h), F32)] + out_shape
    return pl.pallas_call(
        functools.partial(_edge_body, emit_m),
        grid=(nb,),
        in_specs=[
            pl.BlockSpec((be, h), lambda i: (i, 0)),
            pl.BlockSpec((be, h), lambda i: (i, 0)),
            pl.BlockSpec((be, 16), lambda i: (i, 0)),
            pl.BlockSpec((be, ed), lambda i: (i, 0)),
            wspec(1, h), wspec(ed, h), wspec(1, h),
            wspec(h, h), wspec(1, h),
            wspec(h, h), wspec(1, h),
            wspec(1, h), wspec(1, 1),
        ],
        out_specs=out_specs,
        out_shape=out_shape,
    )


def _make_scatter_call(np_, e, w):
    eps = e // NC
    ept = eps // NS
    nchunks = ept // CG
    rpt = np_ // NS
    mesh = plsc.VectorSubcoreMesh(core_axis_name="c", subcore_axis_name="s")

    @functools.partial(
        pl.kernel, mesh=mesh,
        out_type=jax.ShapeDtypeStruct((NC * np_, w), F32),
        scratch_types=[
            pltpu.VMEM((CG,), jnp.int32),
            pltpu.VMEM((CG, w), F32),
            pltpu.VMEM_SHARED((np_, w), F32),
        ],
    )
    def scatter_k(sv, dstv, zt, at, idxv, rows, t_sh):
        cid = lax.axis_index("c")
        sid = lax.axis_index("s")
        zoff = pl.multiple_of(sid * rpt, 8)
        pltpu.sync_copy(zt.at[pl.ds(zoff, rpt)], t_sh.at[pl.ds(zoff, rpt)])
        plsc.subcore_barrier()
        base = pl.multiple_of(cid * eps + sid * ept, 8)

        def chunk(i, carry):
            off = pl.multiple_of(base + i * CG, 8)
            pltpu.sync_copy(dstv.at[pl.ds(off, CG)], idxv)
            pltpu.sync_copy(sv.at[pl.ds(off, CG)], rows)
            pltpu.sync_copy(rows, t_sh.at[idxv], add=True)
            return carry

        lax.fori_loop(0, nchunks, chunk, 0)
        plsc.subcore_barrier()
        woff = pl.multiple_of(cid * np_ + sid * rpt, 8)
        pltpu.sync_copy(t_sh.at[pl.ds(zoff, rpt)], at.at[pl.ds(woff, rpt)])

    return scatter_k


def _make_rc_scatter_call(np_, e):
    eps = e // NC
    ept = eps // NS
    nchunks = ept // CG
    rpt = np_ // NS
    mesh = plsc.VectorSubcoreMesh(core_axis_name="c", subcore_axis_name="s")

    @functools.partial(
        pl.kernel, mesh=mesh,
        compiler_params=pltpu.CompilerParams(needs_layout_passes=False),
        out_type=jax.ShapeDtypeStruct((NC * np_, 128), F32),
        scratch_types=[
            pltpu.VMEM((CG,), jnp.int32),
            pltpu.VMEM((CG, 16), F32),
            pltpu.VMEM((CG, 16), F32),
            pltpu.VMEM((CG, 128), F32),
            pltpu.VMEM_SHARED((np_, 128), F32),
        ],
    )
    def rc_scatter_k(relv, cv, dstv, zt, at, idxv, brl, bc, rows, t_sh):
        cid = lax.axis_index("c")
        sid = lax.axis_index("s")
        zoff = pl.multiple_of(sid * rpt, 8)
        pltpu.sync_copy(zt.at[pl.ds(zoff, rpt)], t_sh.at[pl.ds(zoff, rpt)])
        lane = lax.iota(jnp.int32, 16)
        ehot3 = jnp.where(lane == 3, 1.0, 0.0).astype(F32)
        zero16 = jnp.zeros((16,), F32)

        def initrow(i, carry):
            rows[i, pl.ds(0, 16)] = ehot3
            for j in range(1, 8):
                rows[i, pl.ds(j * 16, 16)] = zero16
            return carry

        lax.fori_loop(0, CG, initrow, 0)
        plsc.subcore_barrier()
        base = pl.multiple_of(cid * eps + sid * ept, 8)
        zeros_i = jnp.zeros((16,), jnp.int32)

        def chunk(i, carry):
            off = pl.multiple_of(base + i * CG, 8)
            pltpu.sync_copy(dstv.at[pl.ds(off, CG)], idxv)
            pltpu.sync_copy(relv.at[pl.ds(off, CG)], brl)
            pltpu.sync_copy(cv.at[pl.ds(off, CG)], bc)

            def group(g, carry2):
                row = g * 16 + lane
                cs = plsc.load_gather(bc, [row, zeros_i])
                for comp in range(3):
                    cvec = jnp.full((16,), comp, jnp.int32)
                    rc = plsc.load_gather(brl, [row, cvec]) * cs
                    plsc.store_scatter(rows, [row, cvec], rc)
                return carry2

            lax.fori_loop(0, CG // 16, group, 0)
            pltpu.sync_copy(rows, t_sh.at[idxv], add=True)
            return carry

        lax.fori_loop(0, nchunks, chunk, 0)
        plsc.subcore_barrier()
        woff = pl.multiple_of(cid * np_ + sid * rpt, 8)
        pltpu.sync_copy(t_sh.at[pl.ds(zoff, rpt)], at.at[pl.ds(woff, rpt)])

    return rc_scatter_k


def _node_body(x_ref, am0_ref, am1_ref, ar0_ref, ar1_ref, pp_ref,
               wnx_ref, wna_ref, bn1_ref, wn2_ref, bn2_ref,
               xn_ref, ppn_ref):
    agg = am0_ref[...] + am1_ref[...]
    r = ar0_ref[...] + ar1_ref[...]
    h = _silu(jnp.dot(x_ref[...], wnx_ref[...], preferred_element_type=F32)
              + jnp.dot(agg, wna_ref[...], preferred_element_type=F32)
              + bn1_ref[...])
    xn_ref[...] = jnp.dot(h, wn2_ref[...], preferred_element_type=F32) \
        + bn2_ref[...]
    lane = lax.broadcasted_iota(jnp.int32, r.shape, 1)
    cnt = jnp.sum(jnp.where(lane == 3, r, 0.0), axis=1, keepdims=True)
    num = jnp.where(lane < 3, r, 0.0)[:, :16]
    ppn_ref[...] = pp_ref[...] + num / jnp.maximum(cnt, 1.0)


def _make_node_call(np_, h, bn):
    nb = np_ // bn
    wspec = lambda r, c: pl.BlockSpec((r, c), lambda i: (0, 0))
    return pl.pallas_call(
        _node_body,
        grid=(nb,),
        in_specs=[
            pl.BlockSpec((bn, h), lambda i: (i, 0)),
            pl.BlockSpec((bn, h), lambda i: (i, 0)),
            pl.BlockSpec((bn, h), lambda i: (nb + i, 0)),
            pl.BlockSpec((bn, 128), lambda i: (i, 0)),
            pl.BlockSpec((bn, 128), lambda i: (nb + i, 0)),
            pl.BlockSpec((bn, 16), lambda i: (i, 0)),
            wspec(h, h), wspec(h, h), wspec(1, h),
            wspec(h, h), wspec(1, h),
        ],
        out_specs=[
            pl.BlockSpec((bn, h), lambda i: (i, 0)),
            pl.BlockSpec((bn, 16), lambda i: (i, 0)),
        ],
        out_shape=[
            jax.ShapeDtypeStruct((np_, h), F32),
            jax.ShapeDtypeStruct((np_, 16), F32),
        ],
    )


def _pos_body(ar0_ref, ar1_ref, pp_ref, ppn_ref):
    r = ar0_ref[...] + ar1_ref[...]
    lane = lax.broadcasted_iota(jnp.int32, r.shape, 1)
    cnt = jnp.sum(jnp.where(lane == 3, r, 0.0), axis=1, keepdims=True)
    num = jnp.where(lane < 3, r, 0.0)[:, :16]
    ppn_ref[...] = pp_ref[...] + num / jnp.maximum(cnt, 1.0)


def _make_pos_call(np_, bn):
    nb = np_ // bn
    return pl.pallas_call(
        _pos_body,
        grid=(nb,),
        in_specs=[
            pl.BlockSpec((bn, 128), lambda i: (i, 0)),
            pl.BlockSpec((bn, 128), lambda i: (nb + i, 0)),
            pl.BlockSpec((bn, 16), lambda i: (i, 0)),
        ],
        out_specs=pl.BlockSpec((bn, 16), lambda i: (i, 0)),
        out_shape=jax.ShapeDtypeStruct((np_, 16), F32),
    )


def kernel(x, pos, edge_index, edge_attr, We1, be1, We2, be2,
           Wn1, bn1, Wn2, bn2, Wc1, bc1, Wc2, bc2):
    n, d = x.shape
    e = edge_index.shape[1]
    ed = edge_attr.shape[1]
    h = We2.shape[2]
    nlayers = We1.shape[0]
    np_ = ((n + 1023) // 1024) * 1024
    bn = 1024
    be = 512

    src = edge_index[0]
    dst = edge_index[1]
    x = jnp.pad(x, ((0, np_ - n), (0, 0)))
    pp = jnp.pad(pos, ((0, np_ - n), (0, 16 - pos.shape[1])))
    zt = jnp.zeros((np_, 128), F32)

    tables_call = _make_tables_call(np_, d, h, bn)
    gather_call = _make_gather_call(np_, e, h)
    edge_call0 = _make_edge_call(e, h, ed, be, True)
    edge_call1 = _make_edge_call(e, h, ed, be, False)
    scatter_call = _make_scatter_call(np_, e, 128)
    rc_scatter_call = _make_rc_scatter_call(np_, e)
    node_call = _make_node_call(np_, h, bn)
    pos_call = _make_pos_call(np_, bn)

    for l in range(nlayers):
        wd = We1[l, 0:d]
        ws = We1[l, d:2 * d]
        wd2 = We1[l, 2 * d:2 * d + 1]
        we = We1[l, 2 * d + 1:]
        td, ts = tables_call(x, wd, ws)
        p4 = pp[:, :4].reshape(np_ * 4)
        gd, gs, grel = gather_call(td, ts, p4, dst, src)
        eargs = (wd2, we, be1[l].reshape(1, h), We2[l],
                 be2[l].reshape(1, h), Wc1[l], bc1[l].reshape(1, h),
                 Wc2[l].reshape(1, h), bc2[l].reshape(1, 1))
        last = l == nlayers - 1
        if not last:
            sm, sr = edge_call0(gd, gs, grel, edge_attr, *eargs)
            am = scatter_call(sm, dst, zt)
            ar = rc_scatter_call(grel, sr, dst, zt)
            x, pp = node_call(x, am, am, ar, ar, pp, Wn1[l, 0:d],
                              Wn1[l, d:], bn1[l].reshape(1, h), Wn2[l],
                              bn2[l].reshape(1, h))
        else:
            sr, = edge_call1(gd, gs, grel, edge_attr, *eargs)
            ar = rc_scatter_call(grel, sr, dst, zt)
            pp = pos_call(ar, ar, pp)
    return pp[:n, :pos.shape[1]]


# speedup vs baseline: 1.4279x; 1.4279x over previous
import functools

import jax
import jax.numpy as jnp
from jax import lax
from jax.experimental import pallas as pl
from jax.experimental.pallas import tpu as pltpu
from jax.experimental.pallas import tpu_sc as plsc

F32 = jnp.float32

NC = 2
NS = 16

CG = 80


def _silu(v):
    return v * jax.nn.sigmoid(v)


def _tables_body(x_ref, wd_ref, ws_ref, td_ref, ts_ref):
    xb = x_ref[...]
    td_ref[...] = jnp.dot(xb, wd_ref[...], preferred_element_type=F32)
    ts_ref[...] = jnp.dot(xb, ws_ref[...], preferred_element_type=F32)


def _make_tables_call(np_, d, h, bn):
    nb = np_ // bn
    return pl.pallas_call(
        _tables_body,
        grid=(nb,),
        in_specs=[
            pl.BlockSpec((bn, d), lambda i: (i, 0)),
            pl.BlockSpec((d, h), lambda i: (0, 0)),
            pl.BlockSpec((d, h), lambda i: (0, 0)),
        ],
        out_specs=[
            pl.BlockSpec((bn, h), lambda i: (i, 0)),
            pl.BlockSpec((bn, h), lambda i: (i, 0)),
        ],
        out_shape=[
            jax.ShapeDtypeStruct((np_, h), F32),
            jax.ShapeDtypeStruct((np_, h), F32),
        ],
    )


def _make_gather_call(np_, e, h, cg):
    epw = e // (NC * NS)
    nchunks = epw // cg
    mesh = plsc.VectorSubcoreMesh(core_axis_name="c", subcore_axis_name="s")

    @functools.partial(
        pl.kernel,
        mesh=mesh,
        compiler_params=pltpu.CompilerParams(needs_layout_passes=False),
        out_type=(
            jax.ShapeDtypeStruct((e, h), F32),
            jax.ShapeDtypeStruct((e, h), F32),
            jax.ShapeDtypeStruct((e, 16), F32),
        ),
        scratch_types=[
            pltpu.VMEM((cg,), jnp.int32),
            pltpu.VMEM((cg,), jnp.int32),
            pltpu.VMEM((cg, h), F32),
            pltpu.VMEM((cg, h), F32),
            pltpu.VMEM((cg, 16), F32),
            pltpu.VMEM((np_ * 4,), F32),
            pltpu.SemaphoreType.DMA,
            pltpu.SemaphoreType.DMA,
        ],
    )
    def gather_k(td, ts, p4, dstv, srcv, gd, gs, grel,
                 idxd, idxs, bxd, bxs, brel, post, s1, s2):
        cid = lax.axis_index("c")
        sid = lax.axis_index("s")
        wid = sid * NC + cid
        base = pl.multiple_of(wid * epw, 8)
        pltpu.sync_copy(p4, post)

        def zrow(i, carry):
            brel[i] = jnp.zeros((16,), F32)
            return carry

        lax.fori_loop(0, cg, zrow, 0)

        def chunk(i, carry):
            off = pl.multiple_of(base + i * cg, 8)
            pltpu.sync_copy(dstv.at[pl.ds(off, cg)], idxd)
            pltpu.sync_copy(srcv.at[pl.ds(off, cg)], idxs)
            a = pltpu.async_copy(td.at[idxd], bxd, s1)
            b = pltpu.async_copy(ts.at[idxs], bxs, s2)

            def group(g, carry2):
                idv = idxd[pl.ds(g * 16, 16)]
                isv = idxs[pl.ds(g * 16, 16)]
                lane = lax.iota(jnp.int32, 16)
                row = g * 16 + lane
                for comp in range(3):
                    cvec = jnp.full((16,), comp, jnp.int32)
                    pd = plsc.load_gather(post, [idv * 4 + comp])
                    ps = plsc.load_gather(post, [isv * 4 + comp])
                    plsc.store_scatter(brel, [row, cvec], pd - ps)
                return carry2

            lax.fori_loop(0, cg // 16, group, 0)
            a.wait()
            b.wait()
            pltpu.sync_copy(bxd, gd.at[pl.ds(off, cg)])
            pltpu.sync_copy(bxs, gs.at[pl.ds(off, cg)])
            pltpu.sync_copy(brel, grel.at[pl.ds(off, cg)])
            return carry

        lax.fori_loop(0, nchunks, chunk, 0)

    return gather_k


def _edge_body(emit_m, gd_ref, gs_ref, rel_ref, ea_ref,
               wd2_ref, we_ref, be1_ref, we2_ref, be2_ref,
               wc1_ref, bc1_ref, wc2_ref, bc2_ref, *out_refs):
    diff = rel_ref[...]
    d2 = jnp.sum(diff * diff, axis=1, keepdims=True)
    pre = (gd_ref[...] + gs_ref[...] + d2 * wd2_ref[...]
           + jnp.dot(ea_ref[...], we_ref[...], preferred_element_type=F32)
           + be1_ref[...])
    m = _silu(pre)
    m2 = _silu(jnp.dot(m, we2_ref[...], preferred_element_type=F32)
               + be2_ref[...])
    cc = _silu(jnp.dot(m2, wc1_ref[...], preferred_element_type=F32)
               + bc1_ref[...])
    c = jnp.sum(cc * wc2_ref[...], axis=1, keepdims=True) + bc2_ref[...]
    diffp = jnp.concatenate(
        [diff, jnp.zeros((diff.shape[0], 128 - diff.shape[1]), F32)], axis=1)
    lane = lax.broadcasted_iota(jnp.int32, diffp.shape, 1)
    count = jnp.where(lane == 3, 1.0, 0.0).astype(F32)
    if emit_m:
        out_refs[0][...] = m2
        out_refs[1][...] = diffp * c + count
    else:
        out_refs[0][...] = diffp * c + count


def _make_edge_call(e, h, ed, be, emit_m):
    nb = e // be
    wspec = lambda r, c: pl.BlockSpec((r, c), lambda i: (0, 0))
    out_specs = [pl.BlockSpec((be, 128), lambda i: (i, 0))]
    out_shape = [jax.ShapeDtypeStruct((e, 128), F32)]
    if emit_m:
        out_specs = [pl.BlockSpec((be, h), lambda i: (i, 0))] + out_specs
        out_shape = [jax.ShapeDtypeStruct((e, h), F32)] + out_shape
    return pl.pallas_call(
        functools.partial(_edge_body, emit_m),
        grid=(nb,),
        in_specs=[
            pl.BlockSpec((be, h), lambda i: (i, 0)),
            pl.BlockSpec((be, h), lambda i: (i, 0)),
            pl.BlockSpec((be, 16), lambda i: (i, 0)),
            pl.BlockSpec((be, ed), lambda i: (i, 0)),
            wspec(1, h), wspec(ed, h), wspec(1, h),
            wspec(h, h), wspec(1, h),
            wspec(h, h), wspec(1, h),
            wspec(1, h), wspec(1, 1),
        ],
        out_specs=out_specs,
        out_shape=out_shape,
    )


def _make_scatter_call(np_, e, w, cg):
    eps = e // NC
    ept = eps // NS
    nchunks = ept // cg
    rpt = np_ // NS
    mesh = plsc.VectorSubcoreMesh(core_axis_name="c", subcore_axis_name="s")

    @functools.partial(
        pl.kernel, mesh=mesh,
        out_type=jax.ShapeDtypeStruct((NC * np_, w), F32),
        scratch_types=[
            pltpu.VMEM((cg,), jnp.int32),
            pltpu.VMEM((cg, w), F32),
            pltpu.VMEM_SHARED((np_, w), F32),
        ],
    )
    def scatter_k(sv, dstv, zt, at, idxv, rows, t_sh):
        cid = lax.axis_index("c")
        sid = lax.axis_index("s")
        zoff = pl.multiple_of(sid * rpt, 8)
        pltpu.sync_copy(zt.at[pl.ds(zoff, rpt)], t_sh.at[pl.ds(zoff, rpt)])
        plsc.subcore_barrier()
        base = pl.multiple_of(cid * eps + sid * ept, 8)

        def chunk(i, carry):
            off = pl.multiple_of(base + i * cg, 8)
            pltpu.sync_copy(dstv.at[pl.ds(off, cg)], idxv)
            pltpu.sync_copy(sv.at[pl.ds(off, cg)], rows)
            pltpu.sync_copy(rows, t_sh.at[idxv], add=True)
            return carry

        lax.fori_loop(0, nchunks, chunk, 0)
        plsc.subcore_barrier()
        woff = pl.multiple_of(cid * np_ + sid * rpt, 8)
        pltpu.sync_copy(t_sh.at[pl.ds(zoff, rpt)], at.at[pl.ds(woff, rpt)])

    return scatter_k


def _node_body(npart, x_ref, *refs):
    am_refs = refs[0:npart]
    ar_refs = refs[npart:2 * npart]
    pp_ref = refs[2 * npart]
    wnx_ref, wna_ref, bn1_ref, wn2_ref, bn2_ref = refs[2 * npart + 1:
                                                       2 * npart + 6]
    xn_ref, ppn_ref = refs[2 * npart + 6:]
    agg = am_refs[0][...]
    r = ar_refs[0][...]
    for j in range(1, npart):
        agg = agg + am_refs[j][...]
        r = r + ar_refs[j][...]
    h = _silu(jnp.dot(x_ref[...], wnx_ref[...], preferred_element_type=F32)
              + jnp.dot(agg, wna_ref[...], preferred_element_type=F32)
              + bn1_ref[...])
    xn_ref[...] = jnp.dot(h, wn2_ref[...], preferred_element_type=F32) \
        + bn2_ref[...]
    lane = lax.broadcasted_iota(jnp.int32, r.shape, 1)
    cnt = jnp.sum(jnp.where(lane == 3, r, 0.0), axis=1, keepdims=True)
    num = jnp.where(lane < 3, r, 0.0)[:, :16]
    ppn_ref[...] = pp_ref[...] + num / jnp.maximum(cnt, 1.0)


def _part_specs(bn, nb, npart):
    specs = []
    for j in range(npart):
        if j % 2 == 0:
            specs.append(pl.BlockSpec((bn, 128), lambda i: (i, 0)))
        else:
            specs.append(
                pl.BlockSpec((bn, 128), lambda i, _nb=nb: (_nb + i, 0)))
    return specs


def _make_node_call(np_, h, bn, npart):
    nb = np_ // bn
    wspec = lambda r, c: pl.BlockSpec((r, c), lambda i: (0, 0))
    return pl.pallas_call(
        functools.partial(_node_body, npart),
        grid=(nb,),
        in_specs=(
            [pl.BlockSpec((bn, h), lambda i: (i, 0))]
            + _part_specs(bn, nb, npart)
            + _part_specs(bn, nb, npart)
            + [pl.BlockSpec((bn, 16), lambda i: (i, 0)),
               wspec(h, h), wspec(h, h), wspec(1, h),
               wspec(h, h), wspec(1, h)]
        ),
        out_specs=[
            pl.BlockSpec((bn, h), lambda i: (i, 0)),
            pl.BlockSpec((bn, 16), lambda i: (i, 0)),
        ],
        out_shape=[
            jax.ShapeDtypeStruct((np_, h), F32),
            jax.ShapeDtypeStruct((np_, 16), F32),
        ],
    )


def _pos_body(npart, *refs):
    ar_refs = refs[0:npart]
    pp_ref = refs[npart]
    ppn_ref = refs[npart + 1]
    r = ar_refs[0][...]
    for j in range(1, npart):
        r = r + ar_refs[j][...]
    lane = lax.broadcasted_iota(jnp.int32, r.shape, 1)
    cnt = jnp.sum(jnp.where(lane == 3, r, 0.0), axis=1, keepdims=True)
    num = jnp.where(lane < 3, r, 0.0)[:, :16]
    ppn_ref[...] = pp_ref[...] + num / jnp.maximum(cnt, 1.0)


def _make_pos_call(np_, bn, npart):
    nb = np_ // bn
    return pl.pallas_call(
        functools.partial(_pos_body, npart),
        grid=(nb,),
        in_specs=(_part_specs(bn, nb, npart)
                  + [pl.BlockSpec((bn, 16), lambda i: (i, 0))]),
        out_specs=pl.BlockSpec((bn, 16), lambda i: (i, 0)),
        out_shape=jax.ShapeDtypeStruct((np_, 16), F32),
    )


def kernel(x, pos, edge_index, edge_attr, We1, be1, We2, be2,
           Wn1, bn1, Wn2, bn2, Wc1, bc1, Wc2, bc2):
    n, d = x.shape
    e = edge_index.shape[1]
    ed = edge_attr.shape[1]
    h = We2.shape[2]
    nlayers = We1.shape[0]
    np_ = ((n + 1023) // 1024) * 1024
    bn = 1024
    be = 640

    grain = NC * NS * CG
    ek0 = ((e // 2) // grain) * grain
    eks = [ek0, e - ek0]
    nk = len(eks)
    src = edge_index[0]
    dst = edge_index[1]
    off = [0, ek0]
    srck = [src[off[k]:off[k] + eks[k]] for k in range(nk)]
    dstk = [dst[off[k]:off[k] + eks[k]] for k in range(nk)]
    eak = [edge_attr[off[k]:off[k] + eks[k]] for k in range(nk)]
    x = jnp.pad(x, ((0, np_ - n), (0, 0)))
    pp = jnp.pad(pos, ((0, np_ - n), (0, 16 - pos.shape[1])))
    zt = jnp.zeros((np_, 128), F32)

    tables_call = _make_tables_call(np_, d, h, bn)
    gather_calls = [_make_gather_call(np_, eks[k], h, CG) for k in range(nk)]
    edge_calls0 = [_make_edge_call(eks[k], h, ed, be, True)
                   for k in range(nk)]
    edge_calls1 = [_make_edge_call(eks[k], h, ed, be, False)
                   for k in range(nk)]
    scatter_calls = [_make_scatter_call(np_, eks[k], 128, CG)
                     for k in range(nk)]
    node_call = _make_node_call(np_, h, bn, 2 * nk)
    pos_call = _make_pos_call(np_, bn, 2 * nk)

    for l in range(nlayers):
        wd = We1[l, 0:d]
        ws = We1[l, d:2 * d]
        wd2 = We1[l, 2 * d:2 * d + 1]
        we = We1[l, 2 * d + 1:]
        td, ts = tables_call(x, wd, ws)
        p4 = pp[:, :4].reshape(np_ * 4)
        eargs = (wd2, we, be1[l].reshape(1, h), We2[l],
                 be2[l].reshape(1, h), Wc1[l], bc1[l].reshape(1, h),
                 Wc2[l].reshape(1, h), bc2[l].reshape(1, 1))
        last = l == nlayers - 1
        ams, ars = [], []
        for k in range(nk):
            gd, gs, grel = gather_calls[k](td, ts, p4, dstk[k], srck[k])
            if not last:
                sm, sr = edge_calls0[k](gd, gs, grel, eak[k], *eargs)
                ams.append(scatter_calls[k](sm, dstk[k], zt))
            else:
                sr, = edge_calls1[k](gd, gs, grel, eak[k], *eargs)
            ars.append(scatter_calls[k](sr, dstk[k], zt))
        amv = [a for a in ams for _ in range(NC)]
        arv = [a for a in ars for _ in range(NC)]
        if not last:
            x, pp = node_call(x, *amv, *arv, pp, Wn1[l, 0:d],
                              Wn1[l, d:], bn1[l].reshape(1, h), Wn2[l],
                              bn2[l].reshape(1, h))
        else:
            pp = pos_call(*arv, pp)
    return pp[:n, :pos.shape[1]]


# speedup vs baseline: 1.5080x; 1.0561x over previous
import functools

import jax
import jax.numpy as jnp
from jax import lax
from jax.experimental import pallas as pl
from jax.experimental.pallas import tpu as pltpu
from jax.experimental.pallas import tpu_sc as plsc

F32 = jnp.float32

NC = 2
NS = 16

CG = 80


def _silu(v):
    return v * jax.nn.sigmoid(v)


def _tables_body(x_ref, wd_ref, ws_ref, td_ref, ts_ref):
    xb = x_ref[...]
    td_ref[...] = jnp.dot(xb, wd_ref[...], preferred_element_type=F32)
    ts_ref[...] = jnp.dot(xb, ws_ref[...], preferred_element_type=F32)


def _make_tables_call(np_, d, h, bn):
    nb = np_ // bn
    return pl.pallas_call(
        _tables_body,
        grid=(nb,),
        in_specs=[
            pl.BlockSpec((bn, d), lambda i: (i, 0)),
            pl.BlockSpec((d, h), lambda i: (0, 0)),
            pl.BlockSpec((d, h), lambda i: (0, 0)),
        ],
        out_specs=[
            pl.BlockSpec((bn, h), lambda i: (i, 0)),
            pl.BlockSpec((bn, h), lambda i: (i, 0)),
        ],
        out_shape=[
            jax.ShapeDtypeStruct((np_, h), F32),
            jax.ShapeDtypeStruct((np_, h), F32),
        ],
    )


def _make_gather_call(np_, e, h, cg):
    epw = e // (NC * NS)
    nchunks = epw // cg
    mesh = plsc.VectorSubcoreMesh(core_axis_name="c", subcore_axis_name="s")

    @functools.partial(
        pl.kernel,
        mesh=mesh,
        compiler_params=pltpu.CompilerParams(needs_layout_passes=False),
        out_type=(
            jax.ShapeDtypeStruct((e, h), F32),
            jax.ShapeDtypeStruct((e, h), F32),
            jax.ShapeDtypeStruct((e, 16), F32),
        ),
        scratch_types=[
            pltpu.VMEM((cg,), jnp.int32),
            pltpu.VMEM((cg,), jnp.int32),
            pltpu.VMEM((cg, h), F32),
            pltpu.VMEM((cg, h), F32),
            pltpu.VMEM((cg, 16), F32),
            pltpu.VMEM((np_ * 4,), F32),
            pltpu.SemaphoreType.DMA,
            pltpu.SemaphoreType.DMA,
        ],
    )
    def gather_k(td, ts, p4, dstv, srcv, gd, gs, grel,
                 idxd, idxs, bxd, bxs, brel, post, s1, s2):
        cid = lax.axis_index("c")
        sid = lax.axis_index("s")
        wid = sid * NC + cid
        base = pl.multiple_of(wid * epw, 8)
        pltpu.sync_copy(p4, post)

        def zrow(i, carry):
            brel[i] = jnp.zeros((16,), F32)
            return carry

        lax.fori_loop(0, cg, zrow, 0)

        def chunk(i, carry):
            off = pl.multiple_of(base + i * cg, 8)
            pltpu.sync_copy(dstv.at[pl.ds(off, cg)], idxd)
            pltpu.sync_copy(srcv.at[pl.ds(off, cg)], idxs)
            a = pltpu.async_copy(td.at[idxd], bxd, s1)
            b = pltpu.async_copy(ts.at[idxs], bxs, s2)

            def group(g, carry2):
                idv = idxd[pl.ds(g * 16, 16)]
                isv = idxs[pl.ds(g * 16, 16)]
                lane = lax.iota(jnp.int32, 16)
                row = g * 16 + lane
                for comp in range(3):
                    cvec = jnp.full((16,), comp, jnp.int32)
                    pd = plsc.load_gather(post, [idv * 4 + comp])
                    ps = plsc.load_gather(post, [isv * 4 + comp])
                    plsc.store_scatter(brel, [row, cvec], pd - ps)
                return carry2

            lax.fori_loop(0, cg // 16, group, 0)
            a.wait()
            b.wait()
            pltpu.sync_copy(bxd, gd.at[pl.ds(off, cg)])
            pltpu.sync_copy(bxs, gs.at[pl.ds(off, cg)])
            pltpu.sync_copy(brel, grel.at[pl.ds(off, cg)])
            return carry

        lax.fori_loop(0, nchunks, chunk, 0)

    return gather_k


def _edge_body(emit_m, gd_ref, gs_ref, rel_ref, ea_ref,
               wd2_ref, we_ref, be1_ref, we2_ref, be2_ref,
               wc1_ref, bc1_ref, wc2_ref, bc2_ref, *out_refs):
    diff = rel_ref[...]
    d2 = jnp.sum(diff * diff, axis=1, keepdims=True)
    pre = (gd_ref[...] + gs_ref[...] + d2 * wd2_ref[...]
           + jnp.dot(ea_ref[...], we_ref[...], preferred_element_type=F32)
           + be1_ref[...])
    m = _silu(pre)
    m2 = _silu(jnp.dot(m, we2_ref[...], preferred_element_type=F32)
               + be2_ref[...])
    cc = _silu(jnp.dot(m2, wc1_ref[...], preferred_element_type=F32)
               + bc1_ref[...])
    c = jnp.sum(cc * wc2_ref[...], axis=1, keepdims=True) + bc2_ref[...]
    diffp = jnp.concatenate(
        [diff, jnp.zeros((diff.shape[0], 128 - diff.shape[1]), F32)], axis=1)
    lane = lax.broadcasted_iota(jnp.int32, diffp.shape, 1)
    count = jnp.where(lane == 3, 1.0, 0.0).astype(F32)
    if emit_m:
        out_refs[0][...] = m2
        out_refs[1][...] = diffp * c + count
    else:
        out_refs[0][...] = diffp * c + count


def _make_edge_call(e, h, ed, be, emit_m):
    nb = e // be
    wspec = lambda r, c: pl.BlockSpec((r, c), lambda i: (0, 0))
    out_specs = [pl.BlockSpec((be, 128), lambda i: (i, 0))]
    out_shape = [jax.ShapeDtypeStruct((e, 128), F32)]
    if emit_m:
        out_specs = [pl.BlockSpec((be, h), lambda i: (i, 0))] + out_specs
        out_shape = [jax.ShapeDtypeStruct((e, h), F32)] + out_shape
    return pl.pallas_call(
        functools.partial(_edge_body, emit_m),
        grid=(nb,),
        in_specs=[
            pl.BlockSpec((be, h), lambda i: (i, 0)),
            pl.BlockSpec((be, h), lambda i: (i, 0)),
            pl.BlockSpec((be, 16), lambda i: (i, 0)),
            pl.BlockSpec((be, ed), lambda i: (i, 0)),
            wspec(1, h), wspec(ed, h), wspec(1, h),
            wspec(h, h), wspec(1, h),
            wspec(h, h), wspec(1, h),
            wspec(1, h), wspec(1, 1),
        ],
        out_specs=out_specs,
        out_shape=out_shape,
    )


def _make_scatter_call(np_, e, w, cg):
    eps = e // NC
    ept = eps // NS
    nchunks = ept // cg
    rpt = np_ // NS
    mesh = plsc.VectorSubcoreMesh(core_axis_name="c", subcore_axis_name="s")

    @functools.partial(
        pl.kernel, mesh=mesh,
        out_type=jax.ShapeDtypeStruct((NC * np_, w), F32),
        scratch_types=[
            pltpu.VMEM((cg,), jnp.int32),
            pltpu.VMEM((cg, w), F32),
            pltpu.VMEM_SHARED((np_, w), F32),
        ],
    )
    def scatter_k(sv, dstv, zt, at, idxv, rows, t_sh):
        cid = lax.axis_index("c")
        sid = lax.axis_index("s")
        zoff = pl.multiple_of(sid * rpt, 8)
        pltpu.sync_copy(zt.at[pl.ds(zoff, rpt)], t_sh.at[pl.ds(zoff, rpt)])
        plsc.subcore_barrier()
        base = pl.multiple_of(cid * eps + sid * ept, 8)

        def chunk(i, carry):
            off = pl.multiple_of(base + i * cg, 8)
            pltpu.sync_copy(dstv.at[pl.ds(off, cg)], idxv)
            pltpu.sync_copy(sv.at[pl.ds(off, cg)], rows)
            pltpu.sync_copy(rows, t_sh.at[idxv], add=True)
            return carry

        lax.fori_loop(0, nchunks, chunk, 0)
        plsc.subcore_barrier()
        woff = pl.multiple_of(cid * np_ + sid * rpt, 8)
        pltpu.sync_copy(t_sh.at[pl.ds(zoff, rpt)], at.at[pl.ds(woff, rpt)])

    return scatter_k


def _node_body(npart, x_ref, *refs):
    am_refs = refs[0:npart]
    ar_refs = refs[npart:2 * npart]
    pp_ref = refs[2 * npart]
    wnx_ref, wna_ref, bn1_ref, wn2_ref, bn2_ref = refs[2 * npart + 1:
                                                       2 * npart + 6]
    xn_ref, ppn_ref = refs[2 * npart + 6:]
    agg = am_refs[0][...]
    r = ar_refs[0][...]
    for j in range(1, npart):
        agg = agg + am_refs[j][...]
        r = r + ar_refs[j][...]
    h = _silu(jnp.dot(x_ref[...], wnx_ref[...], preferred_element_type=F32)
              + jnp.dot(agg, wna_ref[...], preferred_element_type=F32)
              + bn1_ref[...])
    xn_ref[...] = jnp.dot(h, wn2_ref[...], preferred_element_type=F32) \
        + bn2_ref[...]
    lane = lax.broadcasted_iota(jnp.int32, r.shape, 1)
    cnt = jnp.sum(jnp.where(lane == 3, r, 0.0), axis=1, keepdims=True)
    num = jnp.where(lane < 3, r, 0.0)[:, :16]
    ppn_ref[...] = pp_ref[...] + num / jnp.maximum(cnt, 1.0)


def _part_specs(bn, nb, npart):
    specs = []
    for j in range(npart):
        if j % 2 == 0:
            specs.append(pl.BlockSpec((bn, 128), lambda i: (i, 0)))
        else:
            specs.append(
                pl.BlockSpec((bn, 128), lambda i, _nb=nb: (_nb + i, 0)))
    return specs


def _make_node_call(np_, h, bn, npart):
    nb = np_ // bn
    wspec = lambda r, c: pl.BlockSpec((r, c), lambda i: (0, 0))
    return pl.pallas_call(
        functools.partial(_node_body, npart),
        grid=(nb,),
        in_specs=(
            [pl.BlockSpec((bn, h), lambda i: (i, 0))]
            + _part_specs(bn, nb, npart)
            + _part_specs(bn, nb, npart)
            + [pl.BlockSpec((bn, 16), lambda i: (i, 0)),
               wspec(h, h), wspec(h, h), wspec(1, h),
               wspec(h, h), wspec(1, h)]
        ),
        out_specs=[
            pl.BlockSpec((bn, h), lambda i: (i, 0)),
            pl.BlockSpec((bn, 16), lambda i: (i, 0)),
        ],
        out_shape=[
            jax.ShapeDtypeStruct((np_, h), F32),
            jax.ShapeDtypeStruct((np_, 16), F32),
        ],
    )


def _pos_body(npart, *refs):
    ar_refs = refs[0:npart]
    pp_ref = refs[npart]
    ppn_ref = refs[npart + 1]
    r = ar_refs[0][...]
    for j in range(1, npart):
        r = r + ar_refs[j][...]
    lane = lax.broadcasted_iota(jnp.int32, r.shape, 1)
    cnt = jnp.sum(jnp.where(lane == 3, r, 0.0), axis=1, keepdims=True)
    num = jnp.where(lane < 3, r, 0.0)[:, :16]
    ppn_ref[...] = pp_ref[...] + num / jnp.maximum(cnt, 1.0)


def _make_pos_call(np_, bn, npart):
    nb = np_ // bn
    return pl.pallas_call(
        functools.partial(_pos_body, npart),
        grid=(nb,),
        in_specs=(_part_specs(bn, nb, npart)
                  + [pl.BlockSpec((bn, 16), lambda i: (i, 0))]),
        out_specs=pl.BlockSpec((bn, 16), lambda i: (i, 0)),
        out_shape=jax.ShapeDtypeStruct((np_, 16), F32),
    )


def kernel(x, pos, edge_index, edge_attr, We1, be1, We2, be2,
           Wn1, bn1, Wn2, bn2, Wc1, bc1, Wc2, bc2):
    n, d = x.shape
    e = edge_index.shape[1]
    ed = edge_attr.shape[1]
    h = We2.shape[2]
    nlayers = We1.shape[0]
    np_ = ((n + 1023) // 1024) * 1024
    bn = 1024
    be = 640

    grain = NC * NS * CG
    nk = 4
    base_k = ((e // nk) // grain) * grain
    eks = [base_k] * (nk - 1) + [e - base_k * (nk - 1)]
    src = edge_index[0]
    dst = edge_index[1]
    off = [base_k * k for k in range(nk)]
    srck = [src[off[k]:off[k] + eks[k]] for k in range(nk)]
    dstk = [dst[off[k]:off[k] + eks[k]] for k in range(nk)]
    eak = [edge_attr[off[k]:off[k] + eks[k]] for k in range(nk)]
    x = jnp.pad(x, ((0, np_ - n), (0, 0)))
    pp = jnp.pad(pos, ((0, np_ - n), (0, 16 - pos.shape[1])))
    zt = jnp.zeros((np_, 128), F32)

    tables_call = _make_tables_call(np_, d, h, bn)
    gather_calls = [_make_gather_call(np_, eks[k], h, CG) for k in range(nk)]
    edge_calls0 = [_make_edge_call(eks[k], h, ed, be, True)
                   for k in range(nk)]
    edge_calls1 = [_make_edge_call(eks[k], h, ed, be, False)
                   for k in range(nk)]
    scatter_calls = [_make_scatter_call(np_, eks[k], 128, CG)
                     for k in range(nk)]
    node_call = _make_node_call(np_, h, bn, 2 * nk)
    pos_call = _make_pos_call(np_, bn, 2 * nk)

    for l in range(nlayers):
        wd = We1[l, 0:d]
        ws = We1[l, d:2 * d]
        wd2 = We1[l, 2 * d:2 * d + 1]
        we = We1[l, 2 * d + 1:]
        td, ts = tables_call(x, wd, ws)
        p4 = pp[:, :4].reshape(np_ * 4)
        eargs = (wd2, we, be1[l].reshape(1, h), We2[l],
                 be2[l].reshape(1, h), Wc1[l], bc1[l].reshape(1, h),
                 Wc2[l].reshape(1, h), bc2[l].reshape(1, 1))
        last = l == nlayers - 1
        ams, ars = [], []
        for k in range(nk):
            gd, gs, grel = gather_calls[k](td, ts, p4, dstk[k], srck[k])
            if not last:
                sm, sr = edge_calls0[k](gd, gs, grel, eak[k], *eargs)
                ams.append(scatter_calls[k](sm, dstk[k], zt))
            else:
                sr, = edge_calls1[k](gd, gs, grel, eak[k], *eargs)
            ars.append(scatter_calls[k](sr, dstk[k], zt))
        amv = [a for a in ams for _ in range(NC)]
        arv = [a for a in ars for _ in range(NC)]
        if not last:
            x, pp = node_call(x, *amv, *arv, pp, Wn1[l, 0:d],
                              Wn1[l, d:], bn1[l].reshape(1, h), Wn2[l],
                              bn2[l].reshape(1, h))
        else:
            pp = pos_call(*arv, pp)
    return pp[:n, :pos.shape[1]]


# speedup vs baseline: 1.6186x; 1.0733x over previous
import functools

import jax
import jax.numpy as jnp
from jax import lax
from jax.experimental import pallas as pl
from jax.experimental.pallas import tpu as pltpu
from jax.experimental.pallas import tpu_sc as plsc

F32 = jnp.float32

NC = 2
NS = 16

CG = 80


def _silu(v):
    return v * jax.nn.sigmoid(v)


def _tables_body(x_ref, wd_ref, ws_ref, td_ref, ts_ref):
    xb = x_ref[...]
    td_ref[...] = jnp.dot(xb, wd_ref[...], preferred_element_type=F32)
    ts_ref[...] = jnp.dot(xb, ws_ref[...], preferred_element_type=F32)


def _make_tables_call(np_, d, h, bn):
    nb = np_ // bn
    return pl.pallas_call(
        _tables_body,
        grid=(nb,),
        in_specs=[
            pl.BlockSpec((bn, d), lambda i: (i, 0)),
            pl.BlockSpec((d, h), lambda i: (0, 0)),
            pl.BlockSpec((d, h), lambda i: (0, 0)),
        ],
        out_specs=[
            pl.BlockSpec((bn, h), lambda i: (i, 0)),
            pl.BlockSpec((bn, h), lambda i: (i, 0)),
        ],
        out_shape=[
            jax.ShapeDtypeStruct((np_, h), F32),
            jax.ShapeDtypeStruct((np_, h), F32),
        ],
    )


def _make_gather_call(np_, e, h, cg):
    epw = e // (NC * NS)
    nchunks = epw // cg
    mesh = plsc.VectorSubcoreMesh(core_axis_name="c", subcore_axis_name="s")

    npairs = nchunks // 2
    tail = nchunks % 2

    @functools.partial(
        pl.kernel,
        mesh=mesh,
        compiler_params=pltpu.CompilerParams(needs_layout_passes=False),
        out_type=(
            jax.ShapeDtypeStruct((e, h), F32),
            jax.ShapeDtypeStruct((e, h), F32),
            jax.ShapeDtypeStruct((e, 16), F32),
        ),
        scratch_types=[
            pltpu.VMEM((cg,), jnp.int32),
            pltpu.VMEM((cg,), jnp.int32),
            pltpu.VMEM((cg,), jnp.int32),
            pltpu.VMEM((cg,), jnp.int32),
            pltpu.VMEM((cg, h), F32),
            pltpu.VMEM((cg, h), F32),
            pltpu.VMEM((cg, h), F32),
            pltpu.VMEM((cg, h), F32),
            pltpu.VMEM((cg, 16), F32),
            pltpu.VMEM((cg, 16), F32),
            pltpu.VMEM((np_ * 4,), F32),
            pltpu.SemaphoreType.DMA,
            pltpu.SemaphoreType.DMA,
            pltpu.SemaphoreType.DMA,
            pltpu.SemaphoreType.DMA,
            pltpu.SemaphoreType.DMA,
            pltpu.SemaphoreType.DMA,
        ],
    )
    def gather_k(td, ts, p4, dstv, srcv, gd, gs, grel,
                 idxd0, idxs0, idxd1, idxs1, bxd0, bxs0, bxd1, bxs1,
                 brel0, brel1, post,
                 six0, six1, sg0, sg1, so0, so1):
        cid = lax.axis_index("c")
        sid = lax.axis_index("s")
        wid = sid * NC + cid
        base = pl.multiple_of(wid * epw, 8)
        pltpu.sync_copy(p4, post)

        slots = ((idxd0, idxs0, bxd0, bxs0, brel0, six0, sg0, so0),
                 (idxd1, idxs1, bxd1, bxs1, brel1, six1, sg1, so1))

        def start_idx(j, sl):
            off = pl.multiple_of(base + j * cg, 8)
            pltpu.async_copy(dstv.at[pl.ds(off, cg)], sl[0], sl[5])
            pltpu.async_copy(srcv.at[pl.ds(off, cg)], sl[1], sl[5])

        def wait_idx(sl):
            pltpu.make_async_copy(dstv.at[pl.ds(0, cg)], sl[0], sl[5]).wait()
            pltpu.make_async_copy(srcv.at[pl.ds(0, cg)], sl[1], sl[5]).wait()

        def start_gather(sl):
            pltpu.async_copy(td.at[sl[0]], sl[2], sl[6])
            pltpu.async_copy(ts.at[sl[1]], sl[3], sl[6])

        def wait_gather(sl):
            pltpu.make_async_copy(td.at[sl[0]], sl[2], sl[6]).wait()
            pltpu.make_async_copy(ts.at[sl[1]], sl[3], sl[6]).wait()

        def compute_rel(sl):
            idxd, idxs, brel = sl[0], sl[1], sl[4]

            def group(g, carry2):
                idv = idxd[pl.ds(g * 16, 16)]
                isv = idxs[pl.ds(g * 16, 16)]
                lane = lax.iota(jnp.int32, 16)
                row = g * 16 + lane
                for comp in range(3):
                    cvec = jnp.full((16,), comp, jnp.int32)
                    pd = plsc.load_gather(post, [idv * 4 + comp])
                    ps = plsc.load_gather(post, [isv * 4 + comp])
                    plsc.store_scatter(brel, [row, cvec], pd - ps)
                return carry2

            lax.fori_loop(0, cg // 16, group, 0)

        def start_out(j, sl):
            off = pl.multiple_of(base + j * cg, 8)
            pltpu.async_copy(sl[2], gd.at[pl.ds(off, cg)], sl[7])
            pltpu.async_copy(sl[3], gs.at[pl.ds(off, cg)], sl[7])
            pltpu.async_copy(sl[4], grel.at[pl.ds(off, cg)], sl[7])

        def wait_out(sl):
            pltpu.make_async_copy(sl[2], gd.at[pl.ds(0, cg)], sl[7]).wait()
            pltpu.make_async_copy(sl[3], gs.at[pl.ds(0, cg)], sl[7]).wait()
            pltpu.make_async_copy(sl[4], grel.at[pl.ds(0, cg)], sl[7]).wait()

        for sl in slots:

            def zrow(i, carry, _b=sl[4]):
                _b[i] = jnp.zeros((16,), F32)
                return carry

            lax.fori_loop(0, cg, zrow, 0)

        A, B = slots
        start_idx(0, A)
        if nchunks > 1:
            start_idx(1, B)

        def pair(i2, carry):
            c0 = 2 * i2
            wait_idx(A)

            @pl.when(i2 > 0)
            def _():
                wait_out(A)

            start_gather(A)
            compute_rel(A)
            wait_idx(B)

            @pl.when(i2 > 0)
            def _():
                wait_out(B)

            start_gather(B)
            wait_gather(A)
            start_out(c0, A)

            @pl.when(c0 + 2 < nchunks)
            def _():
                start_idx(c0 + 2, A)

            compute_rel(B)
            wait_gather(B)
            start_out(c0 + 1, B)

            @pl.when(c0 + 3 < nchunks)
            def _():
                start_idx(c0 + 3, B)

            return carry

        lax.fori_loop(0, npairs, pair, 0)
        if tail:
            ct = 2 * npairs
            wait_idx(A)
            if npairs > 0:
                wait_out(A)
            start_gather(A)
            compute_rel(A)
            wait_gather(A)
            start_out(ct, A)
        wait_out(A)
        if nchunks > 1:
            wait_out(B)

    return gather_k


def _edge_body(emit_m, gd_ref, gs_ref, rel_ref, ea_ref,
               wd2_ref, we_ref, be1_ref, we2_ref, be2_ref,
               wc1_ref, bc1_ref, wc2_ref, bc2_ref, *out_refs):
    diff = rel_ref[...]
    d2 = jnp.sum(diff * diff, axis=1, keepdims=True)
    pre = (gd_ref[...] + gs_ref[...] + d2 * wd2_ref[...]
           + jnp.dot(ea_ref[...], we_ref[...], preferred_element_type=F32)
           + be1_ref[...])
    m = _silu(pre)
    m2 = _silu(jnp.dot(m, we2_ref[...], preferred_element_type=F32)
               + be2_ref[...])
    cc = _silu(jnp.dot(m2, wc1_ref[...], preferred_element_type=F32)
               + bc1_ref[...])
    c = jnp.sum(cc * wc2_ref[...], axis=1, keepdims=True) + bc2_ref[...]
    diffp = jnp.concatenate(
        [diff, jnp.zeros((diff.shape[0], 128 - diff.shape[1]), F32)], axis=1)
    lane = lax.broadcasted_iota(jnp.int32, diffp.shape, 1)
    count = jnp.where(lane == 3, 1.0, 0.0).astype(F32)
    if emit_m:
        out_refs[0][...] = m2
        out_refs[1][...] = diffp * c + count
    else:
        out_refs[0][...] = diffp * c + count


def _make_edge_call(e, h, ed, be, emit_m):
    nb = e // be
    wspec = lambda r, c: pl.BlockSpec((r, c), lambda i: (0, 0))
    out_specs = [pl.BlockSpec((be, 128), lambda i: (i, 0))]
    out_shape = [jax.ShapeDtypeStruct((e, 128), F32)]
    if emit_m:
        out_specs = [pl.BlockSpec((be, h), lambda i: (i, 0))] + out_specs
        out_shape = [jax.ShapeDtypeStruct((e, h), F32)] + out_shape
    return pl.pallas_call(
        functools.partial(_edge_body, emit_m),
        grid=(nb,),
        in_specs=[
            pl.BlockSpec((be, h), lambda i: (i, 0)),
            pl.BlockSpec((be, h), lambda i: (i, 0)),
            pl.BlockSpec((be, 16), lambda i: (i, 0)),
            pl.BlockSpec((be, ed), lambda i: (i, 0)),
            wspec(1, h), wspec(ed, h), wspec(1, h),
            wspec(h, h), wspec(1, h),
            wspec(h, h), wspec(1, h),
            wspec(1, h), wspec(1, 1),
        ],
        out_specs=out_specs,
        out_shape=out_shape,
    )


def _make_scatter_call(np_, e, w, cg):
    eps = e // NC
    ept = eps // NS
    nchunks = ept // cg
    rpt = np_ // NS
    mesh = plsc.VectorSubcoreMesh(core_axis_name="c", subcore_axis_name="s")

    @functools.partial(
        pl.kernel, mesh=mesh,
        out_type=jax.ShapeDtypeStruct((NC * np_, w), F32),
        scratch_types=[
            pltpu.VMEM((cg,), jnp.int32),
            pltpu.VMEM((cg, w), F32),
            pltpu.VMEM_SHARED((np_, w), F32),
        ],
    )
    def scatter_k(sv, dstv, zt, at, idxv, rows, t_sh):
        cid = lax.axis_index("c")
        sid = lax.axis_index("s")
        zoff = pl.multiple_of(sid * rpt, 8)
        pltpu.sync_copy(zt.at[pl.ds(zoff, rpt)], t_sh.at[pl.ds(zoff, rpt)])
        plsc.subcore_barrier()
        base = pl.multiple_of(cid * eps + sid * ept, 8)

        def chunk(i, carry):
            off = pl.multiple_of(base + i * cg, 8)
            pltpu.sync_copy(dstv.at[pl.ds(off, cg)], idxv)
            pltpu.sync_copy(sv.at[pl.ds(off, cg)], rows)
            pltpu.sync_copy(rows, t_sh.at[idxv], add=True)
            return carry

        lax.fori_loop(0, nchunks, chunk, 0)
        plsc.subcore_barrier()
        woff = pl.multiple_of(cid * np_ + sid * rpt, 8)
        pltpu.sync_copy(t_sh.at[pl.ds(zoff, rpt)], at.at[pl.ds(woff, rpt)])

    return scatter_k


def _node_body(npart, x_ref, *refs):
    am_refs = refs[0:npart]
    ar_refs = refs[npart:2 * npart]
    pp_ref = refs[2 * npart]
    wnx_ref, wna_ref, bn1_ref, wn2_ref, bn2_ref = refs[2 * npart + 1:
                                                       2 * npart + 6]
    xn_ref, ppn_ref = refs[2 * npart + 6:]
    agg = am_refs[0][...]
    r = ar_refs[0][...]
    for j in range(1, npart):
        agg = agg + am_refs[j][...]
        r = r + ar_refs[j][...]
    h = _silu(jnp.dot(x_ref[...], wnx_ref[...], preferred_element_type=F32)
              + jnp.dot(agg, wna_ref[...], preferred_element_type=F32)
              + bn1_ref[...])
    xn_ref[...] = jnp.dot(h, wn2_ref[...], preferred_element_type=F32) \
        + bn2_ref[...]
    lane = lax.broadcasted_iota(jnp.int32, r.shape, 1)
    cnt = jnp.sum(jnp.where(lane == 3, r, 0.0), axis=1, keepdims=True)
    num = jnp.where(lane < 3, r, 0.0)[:, :16]
    ppn_ref[...] = pp_ref[...] + num / jnp.maximum(cnt, 1.0)


def _part_specs(bn, nb, npart):
    specs = []
    for j in range(npart):
        if j % 2 == 0:
            specs.append(pl.BlockSpec((bn, 128), lambda i: (i, 0)))
        else:
            specs.append(
                pl.BlockSpec((bn, 128), lambda i, _nb=nb: (_nb + i, 0)))
    return specs


def _make_node_call(np_, h, bn, npart):
    nb = np_ // bn
    wspec = lambda r, c: pl.BlockSpec((r, c), lambda i: (0, 0))
    return pl.pallas_call(
        functools.partial(_node_body, npart),
        grid=(nb,),
        in_specs=(
            [pl.BlockSpec((bn, h), lambda i: (i, 0))]
            + _part_specs(bn, nb, npart)
            + _part_specs(bn, nb, npart)
            + [pl.BlockSpec((bn, 16), lambda i: (i, 0)),
               wspec(h, h), wspec(h, h), wspec(1, h),
               wspec(h, h), wspec(1, h)]
        ),
        out_specs=[
            pl.BlockSpec((bn, h), lambda i: (i, 0)),
            pl.BlockSpec((bn, 16), lambda i: (i, 0)),
        ],
        out_shape=[
            jax.ShapeDtypeStruct((np_, h), F32),
            jax.ShapeDtypeStruct((np_, 16), F32),
        ],
    )


def _pos_body(npart, *refs):
    ar_refs = refs[0:npart]
    pp_ref = refs[npart]
    ppn_ref = refs[npart + 1]
    r = ar_refs[0][...]
    for j in range(1, npart):
        r = r + ar_refs[j][...]
    lane = lax.broadcasted_iota(jnp.int32, r.shape, 1)
    cnt = jnp.sum(jnp.where(lane == 3, r, 0.0), axis=1, keepdims=True)
    num = jnp.where(lane < 3, r, 0.0)[:, :16]
    ppn_ref[...] = pp_ref[...] + num / jnp.maximum(cnt, 1.0)


def _make_pos_call(np_, bn, npart):
    nb = np_ // bn
    return pl.pallas_call(
        functools.partial(_pos_body, npart),
        grid=(nb,),
        in_specs=(_part_specs(bn, nb, npart)
                  + [pl.BlockSpec((bn, 16), lambda i: (i, 0))]),
        out_specs=pl.BlockSpec((bn, 16), lambda i: (i, 0)),
        out_shape=jax.ShapeDtypeStruct((np_, 16), F32),
    )


def kernel(x, pos, edge_index, edge_attr, We1, be1, We2, be2,
           Wn1, bn1, Wn2, bn2, Wc1, bc1, Wc2, bc2):
    n, d = x.shape
    e = edge_index.shape[1]
    ed = edge_attr.shape[1]
    h = We2.shape[2]
    nlayers = We1.shape[0]
    np_ = ((n + 1023) // 1024) * 1024
    bn = 1024
    be = 640

    grain = NC * NS * CG
    nk = 4
    base_k = ((e // nk) // grain) * grain
    eks = [base_k] * (nk - 1) + [e - base_k * (nk - 1)]
    src = edge_index[0]
    dst = edge_index[1]
    off = [base_k * k for k in range(nk)]
    srck = [src[off[k]:off[k] + eks[k]] for k in range(nk)]
    dstk = [dst[off[k]:off[k] + eks[k]] for k in range(nk)]
    eak = [edge_attr[off[k]:off[k] + eks[k]] for k in range(nk)]
    x = jnp.pad(x, ((0, np_ - n), (0, 0)))
    pp = jnp.pad(pos, ((0, np_ - n), (0, 16 - pos.shape[1])))
    zt = jnp.zeros((np_, 128), F32)

    tables_call = _make_tables_call(np_, d, h, bn)
    gather_calls = [_make_gather_call(np_, eks[k], h, CG) for k in range(nk)]
    edge_calls0 = [_make_edge_call(eks[k], h, ed, be, True)
                   for k in range(nk)]
    edge_calls1 = [_make_edge_call(eks[k], h, ed, be, False)
                   for k in range(nk)]
    scatter_calls = [_make_scatter_call(np_, eks[k], 128, CG)
                     for k in range(nk)]
    node_call = _make_node_call(np_, h, bn, 2 * nk)
    pos_call = _make_pos_call(np_, bn, 2 * nk)

    for l in range(nlayers):
        wd = We1[l, 0:d]
        ws = We1[l, d:2 * d]
        wd2 = We1[l, 2 * d:2 * d + 1]
        we = We1[l, 2 * d + 1:]
        td, ts = tables_call(x, wd, ws)
        p4 = pp[:, :4].reshape(np_ * 4)
        eargs = (wd2, we, be1[l].reshape(1, h), We2[l],
                 be2[l].reshape(1, h), Wc1[l], bc1[l].reshape(1, h),
                 Wc2[l].reshape(1, h), bc2[l].reshape(1, 1))
        last = l == nlayers - 1
        ams, ars = [], []
        for k in range(nk):
            gd, gs, grel = gather_calls[k](td, ts, p4, dstk[k], srck[k])
            if not last:
                sm, sr = edge_calls0[k](gd, gs, grel, eak[k], *eargs)
                ams.append(scatter_calls[k](sm, dstk[k], zt))
            else:
                sr, = edge_calls1[k](gd, gs, grel, eak[k], *eargs)
            ars.append(scatter_calls[k](sr, dstk[k], zt))
        amv = [a for a in ams for _ in range(NC)]
        arv = [a for a in ars for _ in range(NC)]
        if not last:
            x, pp = node_call(x, *amv, *arv, pp, Wn1[l, 0:d],
                              Wn1[l, d:], bn1[l].reshape(1, h), Wn2[l],
                              bn2[l].reshape(1, h))
        else:
            pp = pos_call(*arv, pp)
    return pp[:n, :pos.shape[1]]


# speedup vs baseline: 1.7196x; 1.0624x over previous
import functools

import jax
import jax.numpy as jnp
from jax import lax
from jax.experimental import pallas as pl
from jax.experimental.pallas import tpu as pltpu
from jax.experimental.pallas import tpu_sc as plsc

F32 = jnp.float32

NC = 2
NS = 16

CG = 80


def _silu(v):
    return v * jax.nn.sigmoid(v)


def _tables_body(x_ref, wd_ref, ws_ref, td_ref, ts_ref):
    xb = x_ref[...]
    td_ref[...] = jnp.dot(xb, wd_ref[...], preferred_element_type=F32)
    ts_ref[...] = jnp.dot(xb, ws_ref[...], preferred_element_type=F32)


def _make_tables_call(np_, d, h, bn):
    nb = np_ // bn
    return pl.pallas_call(
        _tables_body,
        grid=(nb,),
        in_specs=[
            pl.BlockSpec((bn, d), lambda i: (i, 0)),
            pl.BlockSpec((d, h), lambda i: (0, 0)),
            pl.BlockSpec((d, h), lambda i: (0, 0)),
        ],
        out_specs=[
            pl.BlockSpec((bn, h), lambda i: (i, 0)),
            pl.BlockSpec((bn, h), lambda i: (i, 0)),
        ],
        out_shape=[
            jax.ShapeDtypeStruct((np_, h), F32),
            jax.ShapeDtypeStruct((np_, h), F32),
        ],
    )


def _make_gather_call(np_, e, h, cg):
    epw = e // (NC * NS)
    nchunks = epw // cg
    mesh = plsc.VectorSubcoreMesh(core_axis_name="c", subcore_axis_name="s")

    npairs = nchunks // 2
    tail = nchunks % 2

    @functools.partial(
        pl.kernel,
        mesh=mesh,
        compiler_params=pltpu.CompilerParams(needs_layout_passes=False),
        out_type=(
            jax.ShapeDtypeStruct((e, h), F32),
            jax.ShapeDtypeStruct((e, h), F32),
            jax.ShapeDtypeStruct((e, 16), F32),
        ),
        scratch_types=[
            pltpu.VMEM((cg,), jnp.int32),
            pltpu.VMEM((cg,), jnp.int32),
            pltpu.VMEM((cg,), jnp.int32),
            pltpu.VMEM((cg,), jnp.int32),
            pltpu.VMEM((cg, h), F32),
            pltpu.VMEM((cg, h), F32),
            pltpu.VMEM((cg, h), F32),
            pltpu.VMEM((cg, h), F32),
            pltpu.VMEM((cg, 16), F32),
            pltpu.VMEM((cg, 16), F32),
            pltpu.VMEM((np_ * 4,), F32),
            pltpu.SemaphoreType.DMA,
            pltpu.SemaphoreType.DMA,
            pltpu.SemaphoreType.DMA,
            pltpu.SemaphoreType.DMA,
            pltpu.SemaphoreType.DMA,
            pltpu.SemaphoreType.DMA,
        ],
    )
    def gather_k(td, ts, p4, dstv, srcv, gd, gs, grel,
                 idxd0, idxs0, idxd1, idxs1, bxd0, bxs0, bxd1, bxs1,
                 brel0, brel1, post,
                 six0, six1, sg0, sg1, so0, so1):
        cid = lax.axis_index("c")
        sid = lax.axis_index("s")
        wid = sid * NC + cid
        base = pl.multiple_of(wid * epw, 8)
        pltpu.sync_copy(p4, post)

        slots = ((idxd0, idxs0, bxd0, bxs0, brel0, six0, sg0, so0),
                 (idxd1, idxs1, bxd1, bxs1, brel1, six1, sg1, so1))

        def start_idx(j, sl):
            off = pl.multiple_of(base + j * cg, 8)
            pltpu.async_copy(dstv.at[pl.ds(off, cg)], sl[0], sl[5])
            pltpu.async_copy(srcv.at[pl.ds(off, cg)], sl[1], sl[5])

        def wait_idx(sl):
            pltpu.make_async_copy(dstv.at[pl.ds(0, cg)], sl[0], sl[5]).wait()
            pltpu.make_async_copy(srcv.at[pl.ds(0, cg)], sl[1], sl[5]).wait()

        def start_gather(sl):
            pltpu.async_copy(td.at[sl[0]], sl[2], sl[6])
            pltpu.async_copy(ts.at[sl[1]], sl[3], sl[6])

        def wait_gather(sl):
            pltpu.make_async_copy(td.at[sl[0]], sl[2], sl[6]).wait()
            pltpu.make_async_copy(ts.at[sl[1]], sl[3], sl[6]).wait()

        def compute_rel(sl):
            idxd, idxs, brel = sl[0], sl[1], sl[4]

            def group(g, carry2):
                idv = idxd[pl.ds(g * 16, 16)]
                isv = idxs[pl.ds(g * 16, 16)]
                lane = lax.iota(jnp.int32, 16)
                row = g * 16 + lane
                for comp in range(3):
                    cvec = jnp.full((16,), comp, jnp.int32)
                    pd = plsc.load_gather(post, [idv * 4 + comp])
                    ps = plsc.load_gather(post, [isv * 4 + comp])
                    plsc.store_scatter(brel, [row, cvec], pd - ps)
                return carry2

            lax.fori_loop(0, cg // 16, group, 0)

        def start_out(j, sl):
            off = pl.multiple_of(base + j * cg, 8)
            pltpu.async_copy(sl[2], gd.at[pl.ds(off, cg)], sl[7])
            pltpu.async_copy(sl[3], gs.at[pl.ds(off, cg)], sl[7])
            pltpu.async_copy(sl[4], grel.at[pl.ds(off, cg)], sl[7])

        def wait_out(sl):
            pltpu.make_async_copy(sl[2], gd.at[pl.ds(0, cg)], sl[7]).wait()
            pltpu.make_async_copy(sl[3], gs.at[pl.ds(0, cg)], sl[7]).wait()
            pltpu.make_async_copy(sl[4], grel.at[pl.ds(0, cg)], sl[7]).wait()

        for sl in slots:

            def zrow(i, carry, _b=sl[4]):
                _b[i] = jnp.zeros((16,), F32)
                return carry

            lax.fori_loop(0, cg, zrow, 0)

        A, B = slots
        start_idx(0, A)
        if nchunks > 1:
            start_idx(1, B)

        def pair(i2, carry):
            c0 = 2 * i2
            wait_idx(A)

            @pl.when(i2 > 0)
            def _():
                wait_out(A)

            start_gather(A)
            compute_rel(A)
            wait_idx(B)

            @pl.when(i2 > 0)
            def _():
                wait_out(B)

            start_gather(B)
            wait_gather(A)
            start_out(c0, A)

            @pl.when(c0 + 2 < nchunks)
            def _():
                start_idx(c0 + 2, A)

            compute_rel(B)
            wait_gather(B)
            start_out(c0 + 1, B)

            @pl.when(c0 + 3 < nchunks)
            def _():
                start_idx(c0 + 3, B)

            return carry

        lax.fori_loop(0, npairs, pair, 0)
        if tail:
            ct = 2 * npairs
            wait_idx(A)
            if npairs > 0:
                wait_out(A)
            start_gather(A)
            compute_rel(A)
            wait_gather(A)
            start_out(ct, A)
        wait_out(A)
        if nchunks > 1:
            wait_out(B)

    return gather_k


def _edge_body(emit_m, gd_ref, gs_ref, rel_ref, ea_ref,
               wd2_ref, we_ref, be1_ref, we2_ref, be2_ref,
               wc1_ref, bc1_ref, wc2_ref, bc2_ref, *out_refs):
    diff = rel_ref[...]
    d2 = jnp.sum(diff * diff, axis=1, keepdims=True)
    pre = (gd_ref[...] + gs_ref[...] + d2 * wd2_ref[...]
           + jnp.dot(ea_ref[...], we_ref[...], preferred_element_type=F32)
           + be1_ref[...])
    m = _silu(pre)
    m2 = _silu(jnp.dot(m, we2_ref[...], preferred_element_type=F32)
               + be2_ref[...])
    cc = _silu(jnp.dot(m2, wc1_ref[...], preferred_element_type=F32)
               + bc1_ref[...])
    c = jnp.sum(cc * wc2_ref[...], axis=1, keepdims=True) + bc2_ref[...]
    diffp = jnp.concatenate(
        [diff, jnp.zeros((diff.shape[0], 128 - diff.shape[1]), F32)], axis=1)
    lane = lax.broadcasted_iota(jnp.int32, diffp.shape, 1)
    count = jnp.where(lane == 3, 1.0, 0.0).astype(F32)
    if emit_m:
        out_refs[0][...] = m2
        out_refs[1][...] = diffp * c + count
    else:
        out_refs[0][...] = diffp * c + count


def _make_edge_call(e, h, ed, be, emit_m):
    nb = e // be
    wspec = lambda r, c: pl.BlockSpec((r, c), lambda i: (0, 0))
    out_specs = [pl.BlockSpec((be, 128), lambda i: (i, 0))]
    out_shape = [jax.ShapeDtypeStruct((e, 128), F32)]
    if emit_m:
        out_specs = [pl.BlockSpec((be, h), lambda i: (i, 0))] + out_specs
        out_shape = [jax.ShapeDtypeStruct((e, h), F32)] + out_shape
    return pl.pallas_call(
        functools.partial(_edge_body, emit_m),
        grid=(nb,),
        in_specs=[
            pl.BlockSpec((be, h), lambda i: (i, 0)),
            pl.BlockSpec((be, h), lambda i: (i, 0)),
            pl.BlockSpec((be, 16), lambda i: (i, 0)),
            pl.BlockSpec((be, ed), lambda i: (i, 0)),
            wspec(1, h), wspec(ed, h), wspec(1, h),
            wspec(h, h), wspec(1, h),
            wspec(h, h), wspec(1, h),
            wspec(1, h), wspec(1, 1),
        ],
        out_specs=out_specs,
        out_shape=out_shape,
    )


def _make_scatter_call(np_, e, w, cg):
    eps = e // NC
    ept = eps // NS
    nchunks = ept // cg
    rpt = np_ // NS
    mesh = plsc.VectorSubcoreMesh(core_axis_name="c", subcore_axis_name="s")

    npairs = nchunks // 2
    tail = nchunks % 2

    @functools.partial(
        pl.kernel, mesh=mesh,
        out_type=jax.ShapeDtypeStruct((NC * np_, w), F32),
        scratch_types=[
            pltpu.VMEM((cg,), jnp.int32),
            pltpu.VMEM((cg,), jnp.int32),
            pltpu.VMEM((cg, w), F32),
            pltpu.VMEM((cg, w), F32),
            pltpu.VMEM_SHARED((np_, w), F32),
            pltpu.SemaphoreType.DMA,
            pltpu.SemaphoreType.DMA,
            pltpu.SemaphoreType.DMA,
            pltpu.SemaphoreType.DMA,
        ],
    )
    def scatter_k(sv, dstv, zt, at, idx0, idx1, rows0, rows1, t_sh,
                  sin0, sin1, ssc0, ssc1):
        cid = lax.axis_index("c")
        sid = lax.axis_index("s")
        zoff = pl.multiple_of(sid * rpt, 8)
        base = pl.multiple_of(cid * eps + sid * ept, 8)
        slots = ((idx0, rows0, sin0, ssc0), (idx1, rows1, sin1, ssc1))

        def start_load(j, sl):
            off = pl.multiple_of(base + j * cg, 8)
            pltpu.async_copy(dstv.at[pl.ds(off, cg)], sl[0], sl[2])
            pltpu.async_copy(sv.at[pl.ds(off, cg)], sl[1], sl[2])

        def wait_load(sl):
            pltpu.make_async_copy(dstv.at[pl.ds(0, cg)], sl[0], sl[2]).wait()
            pltpu.make_async_copy(sv.at[pl.ds(0, cg)], sl[1], sl[2]).wait()

        def start_scat(sl):
            pltpu.async_copy(sl[1], t_sh.at[sl[0]], sl[3], add=True)

        def wait_scat(sl):
            pltpu.make_async_copy(sl[1], t_sh.at[sl[0]], sl[3]).wait()

        A, B = slots
        start_load(0, A)
        if nchunks > 1:
            start_load(1, B)
        pltpu.sync_copy(zt.at[pl.ds(zoff, rpt)], t_sh.at[pl.ds(zoff, rpt)])
        plsc.subcore_barrier()

        def pair(i2, carry):
            c0 = 2 * i2
            wait_load(A)
            start_scat(A)
            wait_load(B)
            start_scat(B)
            wait_scat(A)

            @pl.when(c0 + 2 < nchunks)
            def _():
                start_load(c0 + 2, A)

            wait_scat(B)

            @pl.when(c0 + 3 < nchunks)
            def _():
                start_load(c0 + 3, B)

            return carry

        lax.fori_loop(0, npairs, pair, 0)
        if tail:
            wait_load(A)
            start_scat(A)
            wait_scat(A)
        plsc.subcore_barrier()
        woff = pl.multiple_of(cid * np_ + sid * rpt, 8)
        pltpu.sync_copy(t_sh.at[pl.ds(zoff, rpt)], at.at[pl.ds(woff, rpt)])

    return scatter_k


def _node_body(npart, x_ref, *refs):
    am_refs = refs[0:npart]
    ar_refs = refs[npart:2 * npart]
    pp_ref = refs[2 * npart]
    wnx_ref, wna_ref, bn1_ref, wn2_ref, bn2_ref = refs[2 * npart + 1:
                                                       2 * npart + 6]
    xn_ref, ppn_ref = refs[2 * npart + 6:]
    agg = am_refs[0][...]
    r = ar_refs[0][...]
    for j in range(1, npart):
        agg = agg + am_refs[j][...]
        r = r + ar_refs[j][...]
    h = _silu(jnp.dot(x_ref[...], wnx_ref[...], preferred_element_type=F32)
              + jnp.dot(agg, wna_ref[...], preferred_element_type=F32)
              + bn1_ref[...])
    xn_ref[...] = jnp.dot(h, wn2_ref[...], preferred_element_type=F32) \
        + bn2_ref[...]
    lane = lax.broadcasted_iota(jnp.int32, r.shape, 1)
    cnt = jnp.sum(jnp.where(lane == 3, r, 0.0), axis=1, keepdims=True)
    num = jnp.where(lane < 3, r, 0.0)[:, :16]
    ppn_ref[...] = pp_ref[...] + num / jnp.maximum(cnt, 1.0)


def _part_specs(bn, nb, npart):
    specs = []
    for j in range(npart):
        if j % 2 == 0:
            specs.append(pl.BlockSpec((bn, 128), lambda i: (i, 0)))
        else:
            specs.append(
                pl.BlockSpec((bn, 128), lambda i, _nb=nb: (_nb + i, 0)))
    return specs


def _make_node_call(np_, h, bn, npart):
    nb = np_ // bn
    wspec = lambda r, c: pl.BlockSpec((r, c), lambda i: (0, 0))
    return pl.pallas_call(
        functools.partial(_node_body, npart),
        grid=(nb,),
        in_specs=(
            [pl.BlockSpec((bn, h), lambda i: (i, 0))]
            + _part_specs(bn, nb, npart)
            + _part_specs(bn, nb, npart)
            + [pl.BlockSpec((bn, 16), lambda i: (i, 0)),
               wspec(h, h), wspec(h, h), wspec(1, h),
               wspec(h, h), wspec(1, h)]
        ),
        out_specs=[
            pl.BlockSpec((bn, h), lambda i: (i, 0)),
            pl.BlockSpec((bn, 16), lambda i: (i, 0)),
        ],
        out_shape=[
            jax.ShapeDtypeStruct((np_, h), F32),
            jax.ShapeDtypeStruct((np_, 16), F32),
        ],
    )


def _pos_body(npart, *refs):
    ar_refs = refs[0:npart]
    pp_ref = refs[npart]
    ppn_ref = refs[npart + 1]
    r = ar_refs[0][...]
    for j in range(1, npart):
        r = r + ar_refs[j][...]
    lane = lax.broadcasted_iota(jnp.int32, r.shape, 1)
    cnt = jnp.sum(jnp.where(lane == 3, r, 0.0), axis=1, keepdims=True)
    num = jnp.where(lane < 3, r, 0.0)[:, :16]
    ppn_ref[...] = pp_ref[...] + num / jnp.maximum(cnt, 1.0)


def _make_pos_call(np_, bn, npart):
    nb = np_ // bn
    return pl.pallas_call(
        functools.partial(_pos_body, npart),
        grid=(nb,),
        in_specs=(_part_specs(bn, nb, npart)
                  + [pl.BlockSpec((bn, 16), lambda i: (i, 0))]),
        out_specs=pl.BlockSpec((bn, 16), lambda i: (i, 0)),
        out_shape=jax.ShapeDtypeStruct((np_, 16), F32),
    )


def kernel(x, pos, edge_index, edge_attr, We1, be1, We2, be2,
           Wn1, bn1, Wn2, bn2, Wc1, bc1, Wc2, bc2):
    n, d = x.shape
    e = edge_index.shape[1]
    ed = edge_attr.shape[1]
    h = We2.shape[2]
    nlayers = We1.shape[0]
    np_ = ((n + 1023) // 1024) * 1024
    bn = 1024
    be = 640

    grain = NC * NS * CG
    nk = 4
    base_k = ((e // nk) // grain) * grain
    eks = [base_k] * (nk - 1) + [e - base_k * (nk - 1)]
    src = edge_index[0]
    dst = edge_index[1]
    off = [base_k * k for k in range(nk)]
    srck = [src[off[k]:off[k] + eks[k]] for k in range(nk)]
    dstk = [dst[off[k]:off[k] + eks[k]] for k in range(nk)]
    eak = [edge_attr[off[k]:off[k] + eks[k]] for k in range(nk)]
    x = jnp.pad(x, ((0, np_ - n), (0, 0)))
    pp = jnp.pad(pos, ((0, np_ - n), (0, 16 - pos.shape[1])))
    zt = jnp.zeros((np_, 128), F32)

    tables_call = _make_tables_call(np_, d, h, bn)
    gather_calls = [_make_gather_call(np_, eks[k], h, CG) for k in range(nk)]
    edge_calls0 = [_make_edge_call(eks[k], h, ed, be, True)
                   for k in range(nk)]
    edge_calls1 = [_make_edge_call(eks[k], h, ed, be, False)
                   for k in range(nk)]
    scatter_calls = [_make_scatter_call(np_, eks[k], 128, CG)
                     for k in range(nk)]
    node_call = _make_node_call(np_, h, bn, 2 * nk)
    pos_call = _make_pos_call(np_, bn, 2 * nk)

    for l in range(nlayers):
        wd = We1[l, 0:d]
        ws = We1[l, d:2 * d]
        wd2 = We1[l, 2 * d:2 * d + 1]
        we = We1[l, 2 * d + 1:]
        td, ts = tables_call(x, wd, ws)
        p4 = pp[:, :4].reshape(np_ * 4)
        eargs = (wd2, we, be1[l].reshape(1, h), We2[l],
                 be2[l].reshape(1, h), Wc1[l], bc1[l].reshape(1, h),
                 Wc2[l].reshape(1, h), bc2[l].reshape(1, 1))
        last = l == nlayers - 1
        ams, ars = [], []
        for k in range(nk):
            gd, gs, grel = gather_calls[k](td, ts, p4, dstk[k], srck[k])
            if not last:
                sm, sr = edge_calls0[k](gd, gs, grel, eak[k], *eargs)
                ams.append(scatter_calls[k](sm, dstk[k], zt))
            else:
                sr, = edge_calls1[k](gd, gs, grel, eak[k], *eargs)
            ars.append(scatter_calls[k](sr, dstk[k], zt))
        amv = [a for a in ams for _ in range(NC)]
        arv = [a for a in ars for _ in range(NC)]
        if not last:
            x, pp = node_call(x, *amv, *arv, pp, Wn1[l, 0:d],
                              Wn1[l, d:], bn1[l].reshape(1, h), Wn2[l],
                              bn2[l].reshape(1, h))
        else:
            pp = pos_call(*arv, pp)
    return pp[:n, :pos.shape[1]]
